# baseline (speedup 1.0000x reference)
import jax
import jax.numpy as jnp
from jax.experimental import pallas as pl
from jax.experimental.pallas import tpu as pltpu

_KB = 512


def _fused_kernel(len_ref, c0_ref, c1_ref, feat_ref, out_ref, w_ref):
    L = c0_ref.shape[-1]
    B = out_ref.shape[0]
    KB = feat_ref.shape[2]
    b = pl.program_id(0)
    kb = pl.program_id(1)

    @pl.when((b == 0) & (kb == 0))
    def _():
        out_ref[...] = jnp.zeros_like(out_ref)

    @pl.when(kb == 0)
    def _():
        n = len_ref[b]
        c0 = c0_ref[0]
        c1 = c1_ref[0]
        pos = jax.lax.broadcasted_iota(jnp.int32, (1, L), 1)
        valid = pos < n

        p0 = jnp.roll(c0, 1, axis=1)
        p1 = jnp.roll(c1, 1, axis=1)
        same = (c0 == p0) & (c1 == p1)
        nb = ((pos == 0) | jnp.logical_not(same)) & valid

        s = jnp.where(nb, pos, -1)
        k = 1
        while k < L:
            sh = jnp.where(pos >= k, jnp.roll(s, k, axis=1), -1)
            s = jnp.maximum(s, sh)
            k *= 2

        big = jnp.int32(2**30)
        t = jnp.where(nb, pos, big)
        k = 1
        while k < L:
            sh = jnp.where(pos < L - k, jnp.roll(t, -k, axis=1), big)
            t = jnp.minimum(t, sh)
            k *= 2
        ns = jnp.where(pos < L - 1, jnp.roll(t, -1, axis=1), big)
        ns = jnp.minimum(ns, n)

        run_len = (ns - s).astype(jnp.float32)
        num_groups = jnp.sum(nb.astype(jnp.float32))
        denom = num_groups * run_len
        safe = valid & (denom > 0)
        w_ref[...] = jnp.where(safe, 1.0 / jnp.where(denom > 0, denom, 1.0), 0.0)

    w_chunk = w_ref[:, pl.ds(kb * KB, KB)]
    row = jax.lax.broadcasted_iota(jnp.int32, (B, KB), 0)
    w_rows = jnp.where(row == b, jnp.broadcast_to(w_chunk, (B, KB)), 0.0)
    f = feat_ref[0, 0]
    out_ref[...] += jnp.dot(w_rows, f, preferred_element_type=jnp.float32)


def kernel(input_feature, input_lengths, vq_indices):
    B, N, L, D = input_feature.shape
    c0 = vq_indices[:, :, 0].reshape(B, 1, L).astype(jnp.int32)
    c1 = vq_indices[:, :, 1].reshape(B, 1, L).astype(jnp.int32)
    lengths = input_lengths.astype(jnp.int32)
    nk = L // _KB

    out = pl.pallas_call(
        _fused_kernel,
        grid=(B, nk),
        in_specs=[
            pl.BlockSpec(memory_space=pltpu.SMEM),
            pl.BlockSpec((1, 1, L), lambda b, kb: (b, 0, 0)),
            pl.BlockSpec((1, 1, L), lambda b, kb: (b, 0, 0)),
            pl.BlockSpec((1, 1, _KB, D), lambda b, kb: (b, N - 1, kb, 0)),
        ],
        out_specs=pl.BlockSpec((B, D), lambda b, kb: (0, 0)),
        out_shape=jax.ShapeDtypeStruct((B, D), jnp.float32),
        scratch_shapes=[pltpu.VMEM((1, L), jnp.float32)],
    )(lengths, c0, c1, input_feature)
    return out


# speedup vs baseline: 1.6539x; 1.6539x over previous
import jax
import jax.numpy as jnp
from jax.experimental import pallas as pl
from jax.experimental.pallas import tpu as pltpu

_KB = 2048


def _fused_kernel(len_ref, c0_ref, c1_ref, feat_ref, out_ref, w_ref):
    L = c0_ref.shape[-1]
    B = out_ref.shape[0]
    KB = feat_ref.shape[2]
    b = pl.program_id(0)
    kb = pl.program_id(1)

    @pl.when((b == 0) & (kb == 0))
    def _():
        out_ref[...] = jnp.zeros_like(out_ref)

    @pl.when(kb == 0)
    def _():
        n = len_ref[b]
        c0 = c0_ref[0]
        c1 = c1_ref[0]
        pos = jax.lax.broadcasted_iota(jnp.int32, (1, L), 1)
        valid = pos < n

        p0 = jnp.roll(c0, 1, axis=1)
        p1 = jnp.roll(c1, 1, axis=1)
        same = (c0 == p0) & (c1 == p1)
        nb = ((pos == 0) | jnp.logical_not(same)) & valid

        s = jnp.where(nb, pos, -1)
        k = 1
        while k < L:
            sh = jnp.where(pos >= k, jnp.roll(s, k, axis=1), -1)
            s = jnp.maximum(s, sh)
            k *= 2

        big = jnp.int32(2**30)
        t = jnp.where(nb, pos, big)
        k = 1
        while k < L:
            sh = jnp.where(pos < L - k, jnp.roll(t, -k, axis=1), big)
            t = jnp.minimum(t, sh)
            k *= 2
        ns = jnp.where(pos < L - 1, jnp.roll(t, -1, axis=1), big)
        ns = jnp.minimum(ns, n)

        run_len = (ns - s).astype(jnp.float32)
        num_groups = jnp.sum(nb.astype(jnp.float32))
        denom = num_groups * run_len
        safe = valid & (denom > 0)
        w_ref[...] = jnp.where(safe, 1.0 / jnp.where(denom > 0, denom, 1.0), 0.0)

    w_chunk = w_ref[:, pl.ds(kb * KB, KB)]
    row = jax.lax.broadcasted_iota(jnp.int32, (B, KB), 0)
    w_rows = jnp.where(row == b, jnp.broadcast_to(w_chunk, (B, KB)), 0.0)
    f = feat_ref[0, 0]
    out_ref[...] += jnp.dot(w_rows, f, preferred_element_type=jnp.float32)


def kernel(input_feature, input_lengths, vq_indices):
    B, N, L, D = input_feature.shape
    c0 = vq_indices[:, :, 0].reshape(B, 1, L).astype(jnp.int32)
    c1 = vq_indices[:, :, 1].reshape(B, 1, L).astype(jnp.int32)
    lengths = input_lengths.astype(jnp.int32)
    nk = L // _KB

    out = pl.pallas_call(
        _fused_kernel,
        grid=(B, nk),
        in_specs=[
            pl.BlockSpec(memory_space=pltpu.SMEM),
            pl.BlockSpec((1, 1, L), lambda b, kb: (b, 0, 0)),
            pl.BlockSpec((1, 1, L), lambda b, kb: (b, 0, 0)),
            pl.BlockSpec((1, 1, _KB, D), lambda b, kb: (b, N - 1, kb, 0)),
        ],
        out_specs=pl.BlockSpec((B, D), lambda b, kb: (0, 0)),
        out_shape=jax.ShapeDtypeStruct((B, D), jnp.float32),
        scratch_shapes=[pltpu.VMEM((1, L), jnp.float32)],
    )(lengths, c0, c1, input_feature)
    return out
